# baseline (speedup 1.0000x reference)
import jax
import jax.numpy as jnp
from jax.experimental import pallas as pl
from jax.experimental.pallas import tpu as pltpu

_THETA = 40.0
_KS = (1, 2, 4, 8, 16, 32, 64)


def _build_ladder(d, axis):
    iota = jax.lax.broadcasted_iota(jnp.int32, d.shape, axis)
    dcur = jnp.where(iota == 0, 0.0, d)
    ladder = []
    for k in _KS:
        ladder.append(dcur)
        if k != _KS[-1]:
            dcur = dcur * pltpu.roll(dcur, k, axis)
    return ladder


def _ccedge_body(it_ref, mask_ref, edge_ref, out_ref):
    x0 = mask_ref[...]
    e = jnp.maximum(edge_ref[...], 0.0)
    d = jnp.exp(-_THETA * e)

    lad_h = _build_ladder(d, 2)
    lad_w = _build_ladder(d, 3)
    lad_bh = [pltpu.roll(dk, dk.shape[2] - k, 2) for dk, k in zip(lad_h, _KS)]
    lad_bw = [pltpu.roll(dk, dk.shape[3] - k, 3) for dk, k in zip(lad_w, _KS)]

    def crisscross(x):
        fh, bh, fw, bw = x, x, x, x
        for i, k in enumerate(_KS):
            fh = fh + lad_h[i] * pltpu.roll(fh, k, 2)
            bh = bh + lad_bh[i] * pltpu.roll(bh, bh.shape[2] - k, 2)
            fw = fw + lad_w[i] * pltpu.roll(fw, k, 3)
            bw = bw + lad_bw[i] * pltpu.roll(bw, bw.shape[3] - k, 3)
        return (fh + bh) + (fw + bw) - 3.0 * x

    ones = jnp.ones_like(d)
    rz = 1.0 / crisscross(ones)

    def one_iter(_, x):
        return crisscross(x) * rz

    out_ref[...] = jax.lax.fori_loop(0, it_ref[0], one_iter, x0)


def kernel(mask, edge, iter):
    it = jnp.asarray(iter, jnp.int32).reshape(1)
    return pl.pallas_call(
        _ccedge_body,
        out_shape=jax.ShapeDtypeStruct(mask.shape, mask.dtype),
        in_specs=[
            pl.BlockSpec(memory_space=pltpu.SMEM),
            pl.BlockSpec(memory_space=pltpu.VMEM),
            pl.BlockSpec(memory_space=pltpu.VMEM),
        ],
        out_specs=pl.BlockSpec(memory_space=pltpu.VMEM),
    )(it, mask, edge)
